# Initial kernel scaffold; baseline (speedup 1.0000x reference)
import jax
import jax.numpy as jnp
from jax.experimental import pallas as pl


def kernel(x, pos, edge_index, batch, W1, b1, g1, be1, W2, b2, g2, be2, W3, b3, g3, be3, W4, b4, g4, be4, W5, b5):
    raise NotImplementedError("write your pallas kernel here")


# speedup vs baseline: 1.4177x; 1.4177x over previous
import functools
import jax
import jax.numpy as jnp
from jax import lax
from jax.experimental import pallas as pl
from jax.experimental.pallas import tpu as pltpu

N = 10000
E = 320000
B = 16

_INTERPRET = False


def _k1_body(x_ref, p8_ref, w1xt_ref, w1pt_ref, b1_ref, u_ref, v_ref):
    xw = jnp.dot(x_ref[...], w1xt_ref[...], preferred_element_type=jnp.float32, precision=lax.Precision.HIGHEST)
    pw = jnp.dot(p8_ref[...], w1pt_ref[...], preferred_element_type=jnp.float32, precision=lax.Precision.HIGHEST)
    b1 = b1_ref[0:1, :]
    u_ref[...] = xw + pw + b1
    v_ref[...] = pw


def _k1(x, pos8, w1xt, w1pt8, b1r):
    nb = 1000
    grid = (N // nb,)
    return pl.pallas_call(
        _k1_body,
        grid=grid,
        in_specs=[
            pl.BlockSpec((nb, 128), lambda i: (i, 0)),
            pl.BlockSpec((nb, 8), lambda i: (i, 0)),
            pl.BlockSpec((128, 64), lambda i: (0, 0)),
            pl.BlockSpec((8, 64), lambda i: (0, 0)),
            pl.BlockSpec((8, 64), lambda i: (0, 0)),
        ],
        out_specs=[
            pl.BlockSpec((nb, 64), lambda i: (i, 0)),
            pl.BlockSpec((nb, 64), lambda i: (i, 0)),
        ],
        out_shape=[
            jax.ShapeDtypeStruct((N, 64), jnp.float32),
            jax.ShapeDtypeStruct((N, 64), jnp.float32),
        ],
        interpret=_INTERPRET,
    )(x, pos8, w1xt, w1pt8, b1r)


def _edge_pass1(u, v, src, dst):
    h1 = jax.nn.relu(jnp.take(u, src, axis=0) - jnp.take(v, dst, axis=0))
    s = jnp.sum(h1, axis=0)
    sq = jnp.sum(h1 * h1, axis=0)
    return h1, s, sq


def _k2_body(h1_ref, w_ref, c_ref, h2t_ref, st_ref):
    i = pl.program_id(0)
    h2 = lax.dot_general(w_ref[...], h1_ref[...],
                         (((1,), (1,)), ((), ())),
                         preferred_element_type=jnp.float32,
                         precision=lax.Precision.HIGHEST)
    h2 = jnp.maximum(h2 + c_ref[:, 0:1], 0.0)
    h2t_ref[...] = h2
    rs = jnp.sum(h2, axis=1)
    rq = jnp.sum(h2 * h2, axis=1)
    st = jnp.concatenate([rs[None, :], rq[None, :],
                          jnp.zeros((6, 128), jnp.float32)], axis=0)

    @pl.when(i == 0)
    def _():
        st_ref[...] = jnp.zeros_like(st_ref)

    st_ref[...] += st


def _k2(h1, w2efft, c8):
    eb = 2560
    grid = (E // eb,)
    return pl.pallas_call(
        _k2_body,
        grid=grid,
        in_specs=[
            pl.BlockSpec((eb, 64), lambda i: (i, 0)),
            pl.BlockSpec((128, 64), lambda i: (0, 0)),
            pl.BlockSpec((128, 8), lambda i: (0, 0)),
        ],
        out_specs=[
            pl.BlockSpec((128, eb), lambda i: (0, i)),
            pl.BlockSpec((8, 128), lambda i: (0, 0)),
        ],
        out_shape=[
            jax.ShapeDtypeStruct((128, E), jnp.float32),
            jax.ShapeDtypeStruct((8, 128), jnp.float32),
        ],
        interpret=_INTERPRET,
    )(h1, w2efft, c8)


def _scatter_max(h2t, dst):
    m = jnp.full((N, 128), -1.0, jnp.float32).at[dst].max(h2t.T)
    return m.T


def _k3_body(aggt_ref, p8t_ref, s2_ref, t2_ref, b3_ref, w3a_ref, w3p_ref,
             pt_ref, st_ref):
    aggt = aggt_ref[...]
    aggf = jnp.where(aggt < 0.0, 0.0, s2_ref[:, 0:1] * aggt + t2_ref[:, 0:1])
    p = jnp.dot(w3a_ref[...], aggf, preferred_element_type=jnp.float32, precision=lax.Precision.HIGHEST)
    p = p + jnp.dot(w3p_ref[...], p8t_ref[...], preferred_element_type=jnp.float32, precision=lax.Precision.HIGHEST)
    p = jnp.maximum(p + b3_ref[:, 0:1], 0.0)
    pt_ref[...] = p
    rs = jnp.sum(p, axis=1)
    rq = jnp.sum(p * p, axis=1)
    st_ref[...] = jnp.concatenate([rs[None, :], rq[None, :],
                                   jnp.zeros((6, 128), jnp.float32)], axis=0)


def _k3(aggt, pos8t, s2c, t2c, b3c, w3a, w3p):
    return pl.pallas_call(
        _k3_body,
        in_specs=[
            pl.BlockSpec((128, N), lambda: (0, 0)),
            pl.BlockSpec((8, N), lambda: (0, 0)),
            pl.BlockSpec((128, 8), lambda: (0, 0)),
            pl.BlockSpec((128, 8), lambda: (0, 0)),
            pl.BlockSpec((128, 8), lambda: (0, 0)),
            pl.BlockSpec((128, 128), lambda: (0, 0)),
            pl.BlockSpec((128, 8), lambda: (0, 0)),
        ],
        out_specs=[
            pl.BlockSpec((128, N), lambda: (0, 0)),
            pl.BlockSpec((8, 128), lambda: (0, 0)),
        ],
        out_shape=[
            jax.ShapeDtypeStruct((128, N), jnp.float32),
            jax.ShapeDtypeStruct((8, 128), jnp.float32),
        ],
        interpret=_INTERPRET,
    )(aggt, pos8t, s2c, t2c, b3c, w3a, w3p)


def _k4_body(pt_ref, bm_ref, gm_ref):
    pt = pt_ref[...]
    rows = []
    for b in range(B):
        sel = jnp.where(bm_ref[b:b + 1, :] > 0.5, pt, -1.0)
        rows.append(jnp.max(sel, axis=1)[None, :])
    gm_ref[...] = jnp.concatenate(rows, axis=0)


def _k4(pt, bmask):
    return pl.pallas_call(
        _k4_body,
        in_specs=[
            pl.BlockSpec((128, N), lambda: (0, 0)),
            pl.BlockSpec((B, N), lambda: (0, 0)),
        ],
        out_specs=pl.BlockSpec((B, 128), lambda: (0, 0)),
        out_shape=jax.ShapeDtypeStruct((B, 128), jnp.float32),
        interpret=_INTERPRET,
    )(pt, bmask)


def _k5_body(gm_ref, s3_ref, t3_ref, w4t_ref, b4_ref, g4_ref, be4_ref,
             w5t_ref, b5_ref, out_ref):
    gm = gm_ref[...]
    gfeat = jnp.where(gm < 0.0, 0.0, s3_ref[0:1, :] * gm + t3_ref[0:1, :])
    h4 = jnp.dot(gfeat, w4t_ref[...], preferred_element_type=jnp.float32, precision=lax.Precision.HIGHEST)
    h4 = jnp.maximum(h4 + b4_ref[0:1, :], 0.0)
    mu = jnp.mean(h4, axis=0, keepdims=True)
    var = jnp.mean(h4 * h4, axis=0, keepdims=True) - mu * mu
    h4n = g4_ref[0:1, :] * (h4 - mu) * lax.rsqrt(var + 1e-5) + be4_ref[0:1, :]
    z = jnp.dot(h4n, w5t_ref[...], preferred_element_type=jnp.float32, precision=lax.Precision.HIGHEST) + b5_ref[0:1, :16][:, :10]
    out_ref[...] = jnp.maximum(z, 0.0) + jnp.log1p(jnp.exp(-jnp.abs(z)))


def _k5(gm, s3r, t3r, w4t, b4r, g4r, be4r, w5t, b5r):
    return pl.pallas_call(
        _k5_body,
        in_specs=[pl.BlockSpec(a.shape, lambda: tuple(0 for _ in a.shape))
                  for a in (gm, s3r, t3r, w4t, b4r, g4r, be4r, w5t, b5r)],
        out_specs=pl.BlockSpec((B, 10), lambda: (0, 0)),
        out_shape=jax.ShapeDtypeStruct((B, 10), jnp.float32),
        interpret=_INTERPRET,
    )(gm, s3r, t3r, w4t, b4r, g4r, be4r, w5t, b5r)


def _r8(vec):
    return jnp.broadcast_to(vec[None, :], (8, vec.shape[0]))


def _c8(vec):
    return jnp.broadcast_to(vec[:, None], (vec.shape[0], 8))


def kernel(x, pos, edge_index, batch, W1, b1, g1, be1, W2, b2, g2, be2,
           W3, b3, g3, be3, W4, b4, g4, be4, W5, b5):
    src = edge_index[0]
    dst = edge_index[1]
    pos8 = jnp.pad(pos, ((0, 0), (0, 5)))
    w1xt = W1[:, :128].T
    w1pt8 = jnp.pad(W1[:, 128:].T, ((0, 5), (0, 0)))

    u, v = _k1(x, pos8, w1xt, w1pt8, _r8(b1))

    h1, s1sum, s1sq = _edge_pass1(u, v, src, dst)
    mu1 = s1sum / E
    var1 = s1sq / E - mu1 * mu1
    sc1 = g1 * lax.rsqrt(var1 + 1e-5)
    t1 = be1 - mu1 * sc1
    w2efft = W2 * sc1[None, :]
    c = jnp.dot(t1, W2.T, precision=lax.Precision.HIGHEST) + b2

    h2t, st2 = _k2(h1, w2efft, _c8(c))
    mu2 = st2[0] / E
    var2 = st2[1] / E - mu2 * mu2
    s2 = g2 * lax.rsqrt(var2 + 1e-5)
    t2 = be2 - mu2 * s2

    aggt = _scatter_max(h2t, dst)

    pt, st3 = _k3(aggt, pos8.T, _c8(s2), _c8(t2), _c8(b3), W3[:, :128],
                  jnp.pad(W3[:, 128:], ((0, 0), (0, 5))))
    mu3 = st3[0] / N
    var3 = st3[1] / N - mu3 * mu3
    s3 = g3 * lax.rsqrt(var3 + 1e-5)
    t3 = be3 - mu3 * s3

    bmask = (batch[None, :] == jnp.arange(B, dtype=jnp.int32)[:, None]).astype(jnp.float32)
    gm = _k4(pt, bmask)

    out = _k5(gm, _r8(s3), _r8(t3), W4.T, _r8(b4), _r8(g4), _r8(be4),
              W5.T, _r8(jnp.pad(b5, (0, 6))))
    return out


# speedup vs baseline: 1.6213x; 1.1436x over previous
import functools
import jax
import jax.numpy as jnp
from jax import lax
from jax.experimental import pallas as pl
from jax.experimental.pallas import tpu as pltpu
from jax.experimental.pallas import tpu_sc as plsc

try:
    _SC_INFO = plsc.get_sparse_core_info()
    _NC = _SC_INFO.num_cores
    _NS = _SC_INFO.num_subcores
except Exception:
    _NC, _NS = 2, 16
_NW = _NC * _NS

N = 10000
E = 320000
B = 16

_INTERPRET = False

_EB = 2560
_NEB = E // _EB


def _k1_body(x_ref, w1xt_ref, b1_ref, u_ref):
    xw = jnp.dot(x_ref[...], w1xt_ref[...], preferred_element_type=jnp.float32)
    u_ref[...] = xw + b1_ref[0:1, :]


def _k1(x, w1xt, b1r):
    nb = 1000
    return pl.pallas_call(
        _k1_body,
        grid=(N // nb,),
        in_specs=[
            pl.BlockSpec((nb, 128), lambda i: (i, 0)),
            pl.BlockSpec((128, 64), lambda i: (0, 0)),
            pl.BlockSpec((8, 64), lambda i: (0, 0)),
        ],
        out_specs=pl.BlockSpec((nb, 64), lambda i: (i, 0)),
        out_shape=jax.ShapeDtypeStruct((N, 64), jnp.float32),
        interpret=_INTERPRET,
    )(x, w1xt, b1r)


_ROWS = E // 128


def _sc1_body(u_hbm, p16_hbm, src_hbm, dst_hbm, ug_hbm, dp_hbm,
              idx_s, idx_d, ubuf, psbuf, pdbuf, sem_u, sem_s, sem_d):
    wid = lax.axis_index("s") * _NC + lax.axis_index("c")
    nrows = (_ROWS - wid + _NW - 1) // _NW

    def row_body(j, _):
        r = wid + j * _NW
        pltpu.sync_copy(src_hbm.at[r], idx_s)
        pltpu.sync_copy(dst_hbm.at[r], idx_d)
        cu = pltpu.async_copy(u_hbm.at[idx_s], ubuf, sem_u)
        cs = pltpu.async_copy(p16_hbm.at[idx_s], psbuf, sem_s)
        cd = pltpu.async_copy(p16_hbm.at[idx_d], pdbuf, sem_d)
        cu.wait()
        cs.wait()
        cd.wait()

        def e_body(e, _c):
            psbuf[e, :] = psbuf[e, :] - pdbuf[e, :]
            return 0

        lax.fori_loop(0, 128, e_body, 0)
        pltpu.sync_copy(ubuf, ug_hbm.at[pl.ds(r * 128, 128)])
        pltpu.sync_copy(psbuf, dp_hbm.at[pl.ds(r * 128, 128)])
        return 0

    lax.fori_loop(0, nrows, row_body, 0)


def _edge_pass1(u, pos16, src, dst):
    src2d = src.reshape(_ROWS, 128)
    dst2d = dst.reshape(_ROWS, 128)
    mesh = plsc.VectorSubcoreMesh(core_axis_name="c", subcore_axis_name="s")
    f = pl.kernel(
        _sc1_body,
        mesh=mesh,
        out_type=[
            jax.ShapeDtypeStruct((E, 64), jnp.float32),
            jax.ShapeDtypeStruct((E, 16), jnp.float32),
        ],
        scratch_types=[
            pltpu.VMEM((128,), jnp.int32),
            pltpu.VMEM((128,), jnp.int32),
            pltpu.VMEM((128, 64), jnp.float32),
            pltpu.VMEM((128, 16), jnp.float32),
            pltpu.VMEM((128, 16), jnp.float32),
            pltpu.SemaphoreType.DMA,
            pltpu.SemaphoreType.DMA,
            pltpu.SemaphoreType.DMA,
        ],
        compiler_params=pltpu.CompilerParams(use_tc_tiling_on_sc=False),
    )
    return f(u, pos16, src2d, dst2d)


def _h1_block(ug, dp, w1p16):
    pw = jnp.dot(dp, w1p16, preferred_element_type=jnp.float32)
    return jnp.maximum(ug + pw, 0.0)


def _k1b_body(ug_ref, dp_ref, w1p_ref, st_ref):
    h1 = _h1_block(ug_ref[...], dp_ref[...], w1p_ref[...])
    mean = jnp.mean(h1, axis=0, keepdims=True)
    d = h1 - mean
    m2 = jnp.sum(d * d, axis=0, keepdims=True)
    st_ref[...] = jnp.concatenate(
        [mean, m2, jnp.zeros((6, 64), jnp.float32)], axis=0)


def _k1b(ug, dp, w1p16):
    return pl.pallas_call(
        _k1b_body,
        grid=(_NEB,),
        in_specs=[
            pl.BlockSpec((_EB, 64), lambda i: (i, 0)),
            pl.BlockSpec((_EB, 16), lambda i: (i, 0)),
            pl.BlockSpec((16, 64), lambda i: (0, 0)),
        ],
        out_specs=pl.BlockSpec((8, 64), lambda i: (i, 0)),
        out_shape=jax.ShapeDtypeStruct((_NEB * 8, 64), jnp.float32),
        interpret=_INTERPRET,
    )(ug, dp, w1p16)


def _chan_combine(st, nfeat, nblk, per_blk):
    means = st.reshape(nblk, 8, nfeat)[:, 0, :]
    m2s = st.reshape(nblk, 8, nfeat)[:, 1, :]
    mu = jnp.mean(means, axis=0)
    m2 = jnp.sum(m2s, axis=0) + per_blk * jnp.sum((means - mu[None, :]) ** 2, axis=0)
    var = m2 / (nblk * per_blk)
    return mu, var


def _k2_body(ug_ref, dp_ref, w1p_ref, mu1_ref, sd1_ref, g1_ref, be1_ref,
             w2_ref, b2_ref, h2t_ref, st_ref):
    h1 = _h1_block(ug_ref[...], dp_ref[...], w1p_ref[...])
    h1n = g1_ref[0:1, :] * (h1 - mu1_ref[0:1, :]) / sd1_ref[0:1, :] + be1_ref[0:1, :]
    h2 = lax.dot_general(w2_ref[...], h1n, (((1,), (1,)), ((), ())),
                         preferred_element_type=jnp.float32)
    h2 = jnp.maximum(h2 + b2_ref[:, 0:1], 0.0)
    h2t_ref[...] = h2
    mean = jnp.mean(h2, axis=1, keepdims=True)
    d = h2 - mean
    m2 = jnp.sum(d * d, axis=1, keepdims=True)
    st_ref[...] = jnp.concatenate(
        [mean.T, m2.T, jnp.zeros((6, 128), jnp.float32)], axis=0)


def _k2(ug, dp, w1p16, mu1r, sd1r, g1r, be1r, w2, b2c):
    return pl.pallas_call(
        _k2_body,
        grid=(_NEB,),
        in_specs=[
            pl.BlockSpec((_EB, 64), lambda i: (i, 0)),
            pl.BlockSpec((_EB, 16), lambda i: (i, 0)),
            pl.BlockSpec((16, 64), lambda i: (0, 0)),
            pl.BlockSpec((8, 64), lambda i: (0, 0)),
            pl.BlockSpec((8, 64), lambda i: (0, 0)),
            pl.BlockSpec((8, 64), lambda i: (0, 0)),
            pl.BlockSpec((8, 64), lambda i: (0, 0)),
            pl.BlockSpec((128, 64), lambda i: (0, 0)),
            pl.BlockSpec((128, 8), lambda i: (0, 0)),
        ],
        out_specs=[
            pl.BlockSpec((128, _EB), lambda i: (0, i)),
            pl.BlockSpec((8, 128), lambda i: (i, 0)),
        ],
        out_shape=[
            jax.ShapeDtypeStruct((128, E), jnp.float32),
            jax.ShapeDtypeStruct((_NEB * 8, 128), jnp.float32),
        ],
        interpret=_INTERPRET,
    )(ug, dp, w1p16, mu1r, sd1r, g1r, be1r, w2, b2c)


_SM_CHUNK = 1280
_SM_NCH = E // _SM_CHUNK


def _sc3_body(h2t_hbm, dst_hbm, aggt_hbm,
              dstbuf, vb0, vb1, vb2, vb3, acc0, acc1, acc2, acc3, tmp):
    wid = lax.axis_index("s") * _NC + lax.axis_index("c")
    f0 = wid * 4
    accs = (acc0, acc1, acc2, acc3)
    vbs = (vb0, vb1, vb2, vb3)

    def init_body(i, _):
        neg = jnp.full((16,), -1.0, jnp.float32)
        for q in range(4):
            accs[q][pl.ds(i * 16, 16)] = neg
        return 0

    lax.fori_loop(0, N // 16, init_body, 0)

    ids = lax.iota(jnp.int32, 16)

    def chunk_body(k, _):
        base = k * _SM_CHUNK
        pltpu.sync_copy(dst_hbm.at[pl.ds(base, _SM_CHUNK)], dstbuf)
        for q in range(4):
            pltpu.sync_copy(h2t_hbm.at[f0 + q, pl.ds(base, _SM_CHUNK)], vbs[q])

        def group_body(g, _):
            idx = dstbuf[pl.ds(g * 16, 16)]
            vals = [vbs[q][pl.ds(g * 16, 16)] for q in range(4)]

            def cond(act):
                return jnp.max(act, axis=0) > 0

            def body(act):
                m = act > 0
                plsc.store_scatter(tmp, [idx], ids, mask=m)
                rr = plsc.load_gather(tmp, [idx])
                win = m & (rr == ids)
                for q in range(4):
                    old = plsc.load_gather(accs[q], [idx])
                    plsc.store_scatter(accs[q], [idx],
                                       jnp.maximum(old, vals[q]), mask=win)
                return jnp.where(win, 0, act.astype(jnp.int32))

            lax.while_loop(cond, body, jnp.ones((16,), jnp.int32))
            return 0

        lax.fori_loop(0, _SM_CHUNK // 16, group_body, 0)
        return 0

    lax.fori_loop(0, _SM_NCH, chunk_body, 0)
    for q in range(4):
        pltpu.sync_copy(accs[q], aggt_hbm.at[f0 + q])


def _scatter_max(h2t, dst):
    mesh = plsc.VectorSubcoreMesh(core_axis_name="c", subcore_axis_name="s")
    f = pl.kernel(
        _sc3_body,
        mesh=mesh,
        out_type=jax.ShapeDtypeStruct((128, N), jnp.float32),
        scratch_types=[
            pltpu.VMEM((_SM_CHUNK,), jnp.int32),
            pltpu.VMEM((_SM_CHUNK,), jnp.float32),
            pltpu.VMEM((_SM_CHUNK,), jnp.float32),
            pltpu.VMEM((_SM_CHUNK,), jnp.float32),
            pltpu.VMEM((_SM_CHUNK,), jnp.float32),
            pltpu.VMEM((N,), jnp.float32),
            pltpu.VMEM((N,), jnp.float32),
            pltpu.VMEM((N,), jnp.float32),
            pltpu.VMEM((N,), jnp.float32),
            pltpu.VMEM((N,), jnp.int32),
        ],
        compiler_params=pltpu.CompilerParams(needs_layout_passes=False),
    )
    return f(h2t, dst)


def _k3_body(aggt_ref, p16t_ref, mu2_ref, sd2_ref, g2_ref, be2_ref,
             w3a_ref, w3p_ref, b3_ref, pt_ref, st_ref):
    aggt = aggt_ref[...]
    aggf = jnp.where(aggt < 0.0, 0.0,
                     g2_ref[:, 0:1] * (aggt - mu2_ref[:, 0:1]) / sd2_ref[:, 0:1]
                     + be2_ref[:, 0:1])
    p = jnp.dot(w3a_ref[...], aggf, preferred_element_type=jnp.float32)
    p = p + jnp.dot(w3p_ref[...], p16t_ref[...], preferred_element_type=jnp.float32)
    p = jnp.maximum(p + b3_ref[:, 0:1], 0.0)
    pt_ref[...] = p
    mean = jnp.mean(p, axis=1, keepdims=True)
    d = p - mean
    var = jnp.mean(d * d, axis=1, keepdims=True)
    st_ref[...] = jnp.concatenate(
        [mean.T, var.T, jnp.zeros((6, 128), jnp.float32)], axis=0)


def _k3(aggt, pos16t, mu2c, sd2c, g2c, be2c, w3a, w3p16, b3c):
    return pl.pallas_call(
        _k3_body,
        in_specs=[
            pl.BlockSpec((128, N), lambda: (0, 0)),
            pl.BlockSpec((16, N), lambda: (0, 0)),
            pl.BlockSpec((128, 8), lambda: (0, 0)),
            pl.BlockSpec((128, 8), lambda: (0, 0)),
            pl.BlockSpec((128, 8), lambda: (0, 0)),
            pl.BlockSpec((128, 8), lambda: (0, 0)),
            pl.BlockSpec((128, 128), lambda: (0, 0)),
            pl.BlockSpec((128, 16), lambda: (0, 0)),
            pl.BlockSpec((128, 8), lambda: (0, 0)),
        ],
        out_specs=[
            pl.BlockSpec((128, N), lambda: (0, 0)),
            pl.BlockSpec((8, 128), lambda: (0, 0)),
        ],
        out_shape=[
            jax.ShapeDtypeStruct((128, N), jnp.float32),
            jax.ShapeDtypeStruct((8, 128), jnp.float32),
        ],
        interpret=_INTERPRET,
    )(aggt, pos16t, mu2c, sd2c, g2c, be2c, w3a, w3p16, b3c)


def _k4_body(pt_ref, bm_ref, gm_ref):
    pt = pt_ref[...]
    rows = []
    for b in range(B):
        sel = jnp.where(bm_ref[b:b + 1, :] > 0.5, pt, -1.0)
        rows.append(jnp.max(sel, axis=1)[None, :])
    gm_ref[...] = jnp.concatenate(rows, axis=0)


def _k4(pt, bmask):
    return pl.pallas_call(
        _k4_body,
        in_specs=[
            pl.BlockSpec((128, N), lambda: (0, 0)),
            pl.BlockSpec((B, N), lambda: (0, 0)),
        ],
        out_specs=pl.BlockSpec((B, 128), lambda: (0, 0)),
        out_shape=jax.ShapeDtypeStruct((B, 128), jnp.float32),
        interpret=_INTERPRET,
    )(pt, bmask)


def _k5_body(gm_ref, mu3_ref, sd3_ref, g3_ref, be3_ref, w4t_ref, b4_ref,
             g4_ref, be4_ref, w5t_ref, b5_ref, out_ref):
    gm = gm_ref[...]
    gfeat = jnp.where(gm < 0.0, 0.0,
                      g3_ref[0:1, :] * (gm - mu3_ref[0:1, :]) / sd3_ref[0:1, :]
                      + be3_ref[0:1, :])
    h4 = jnp.dot(gfeat, w4t_ref[...], preferred_element_type=jnp.float32)
    h4 = jnp.maximum(h4 + b4_ref[0:1, :], 0.0)
    mu = jnp.mean(h4, axis=0, keepdims=True)
    d = h4 - mu
    var = jnp.mean(d * d, axis=0, keepdims=True)
    h4n = g4_ref[0:1, :] * d / jnp.sqrt(var + 1e-5) + be4_ref[0:1, :]
    z = jnp.dot(h4n, w5t_ref[...], preferred_element_type=jnp.float32) + b5_ref[0:1, :10]
    out_ref[...] = jnp.maximum(z, 0.0) + jnp.log1p(jnp.exp(-jnp.abs(z)))


def _k5(gm, mu3r, sd3r, g3r, be3r, w4t, b4r, g4r, be4r, w5t, b5r):
    args = (gm, mu3r, sd3r, g3r, be3r, w4t, b4r, g4r, be4r, w5t, b5r)
    return pl.pallas_call(
        _k5_body,
        in_specs=[pl.BlockSpec(a.shape, lambda: (0, 0)) for a in args],
        out_specs=pl.BlockSpec((B, 10), lambda: (0, 0)),
        out_shape=jax.ShapeDtypeStruct((B, 10), jnp.float32),
        interpret=_INTERPRET,
    )(*args)


def _r8(vec):
    return jnp.broadcast_to(vec[None, :], (8, vec.shape[0]))


def _c8(vec):
    return jnp.broadcast_to(vec[:, None], (vec.shape[0], 8))


def kernel(x, pos, edge_index, batch, W1, b1, g1, be1, W2, b2, g2, be2,
           W3, b3, g3, be3, W4, b4, g4, be4, W5, b5):
    src = edge_index[0]
    dst = edge_index[1]
    pos16 = jnp.pad(pos, ((0, 0), (0, 13)))
    w1xt = W1[:, :128].T
    w1p16 = jnp.pad(W1[:, 128:].T, ((0, 13), (0, 0)))

    u = _k1(x, w1xt, _r8(b1))
    ug, dp = _edge_pass1(u, pos16, src, dst)

    st1 = _k1b(ug, dp, w1p16)
    mu1, var1 = _chan_combine(st1, 64, _NEB, _EB)
    sd1 = jnp.sqrt(var1 + 1e-5)

    h2t, st2 = _k2(ug, dp, w1p16, _r8(mu1), _r8(sd1), _r8(g1), _r8(be1),
                   W2, _c8(b2))
    mu2, var2 = _chan_combine(st2, 128, _NEB, _EB)
    sd2 = jnp.sqrt(var2 + 1e-5)

    aggt = _scatter_max(h2t, dst)

    pt, st3 = _k3(aggt, pos16.T, _c8(mu2), _c8(sd2), _c8(g2), _c8(be2),
                  W3[:, :128], jnp.pad(W3[:, 128:], ((0, 0), (0, 13))), _c8(b3))
    mu3 = st3[0]
    sd3 = jnp.sqrt(st3[1] + 1e-5)

    bmask = (batch[None, :] == jnp.arange(B, dtype=jnp.int32)[:, None]).astype(jnp.float32)
    gm = _k4(pt, bmask)

    out = _k5(gm, _r8(mu3), _r8(sd3), _r8(g3), _r8(be3), W4.T, _r8(b4),
              _r8(g4), _r8(be4), W5.T, _r8(jnp.pad(b5, (0, 6))))
    return out
